# baseline (speedup 1.0000x reference)
import jax
import jax.numpy as jnp
from jax.experimental import pallas as pl
from jax.experimental.pallas import tpu as pltpu

_NUM_CLASSES = 100


def _ru(x, m):
    return ((x + m - 1) // m) * m


def _conv_flat(x_flat, w_taps, bias, *, Wp, k, pad, S, tm_target, mask):
    R, Cin = x_flat.shape
    T, _, Cout = w_taps.shape
    offsets = [dh * Wp + dw for dh in range(k) for dw in range(k)]
    D = pad * Wp + pad
    tm = min(tm_target, _ru(R, 8))
    nb = -(-R // tm)
    Rp = nb * tm
    ext_len = (nb + 1) * tm
    x_ext = jnp.pad(x_flat, ((D, ext_len - D - R), (0, 0)))
    halo = _ru(max(offsets), 8) if max(offsets) else 0
    Hp = S // Wp

    def body(x0_ref, x1_ref, w_ref, b_ref, o_ref):
        if halo:
            xcat = jnp.concatenate([x0_ref[...], x1_ref[:halo]], axis=0)
        else:
            xcat = x0_ref[...]
        acc = jnp.zeros((tm, Cout), jnp.float32)
        for t, off in enumerate(offsets):
            acc = acc + jnp.dot(
                xcat[off:off + tm], w_ref[t], preferred_element_type=jnp.float32
            )
        r = jnp.maximum(acc + b_ref[...], 0.0)
        if mask:
            row = pl.program_id(0) * tm + jax.lax.broadcasted_iota(
                jnp.int32, (tm, 1), 0
            )
            s = jax.lax.rem(row, S)
            h = jax.lax.div(s, Wp)
            w = jax.lax.rem(s, Wp)
            ok = (h >= pad) & (h < Hp - pad) & (w >= pad) & (w < Wp - pad)
            r = jnp.where(ok, r, 0.0)
        o_ref[...] = r.astype(o_ref.dtype)

    return pl.pallas_call(
        body,
        out_shape=jax.ShapeDtypeStruct((Rp, Cout), jnp.bfloat16),
        grid=(nb,),
        in_specs=[
            pl.BlockSpec((tm, Cin), lambda i: (i, 0)),
            pl.BlockSpec((tm, Cin), lambda i: (i + 1, 0)),
            pl.BlockSpec((T, Cin, Cout), lambda i: (0, 0, 0)),
            pl.BlockSpec((1, Cout), lambda i: (0, 0)),
        ],
        out_specs=pl.BlockSpec((tm, Cout), lambda i: (i, 0)),
        compiler_params=pltpu.CompilerParams(
            dimension_semantics=("parallel",),
            vmem_limit_bytes=48 * 1024 * 1024,
        ),
    )(x_ext, x_ext, w_taps, bias)


def _pool_flat(x_flat, *, Wp, tm_target):
    R, C = x_flat.shape
    offsets = [dh * Wp + dw for dh in range(3) for dw in range(3)]
    tm = min(tm_target, _ru(R, 8))
    nb = -(-R // tm)
    Rp = nb * tm
    ext_len = (nb + 1) * tm
    x_ext = jnp.pad(x_flat, ((0, ext_len - R), (0, 0)))
    halo = _ru(max(offsets), 8)

    def body(x0_ref, x1_ref, o_ref):
        xcat = jnp.concatenate([x0_ref[...], x1_ref[:halo]], axis=0)
        r = xcat[0:tm]
        for off in offsets[1:]:
            r = jnp.maximum(r, xcat[off:off + tm])
        o_ref[...] = r

    return pl.pallas_call(
        body,
        out_shape=jax.ShapeDtypeStruct((Rp, C), x_flat.dtype),
        grid=(nb,),
        in_specs=[
            pl.BlockSpec((tm, C), lambda i: (i, 0)),
            pl.BlockSpec((tm, C), lambda i: (i + 1, 0)),
        ],
        out_specs=pl.BlockSpec((tm, C), lambda i: (i, 0)),
        compiler_params=pltpu.CompilerParams(
            dimension_semantics=("parallel",),
            vmem_limit_bytes=48 * 1024 * 1024,
        ),
    )(x_ext, x_ext)


def _fc(a, w, bias, *, tn, tk, relu, n_out, out_dtype):
    M, K = a.shape
    Kp, Np = w.shape
    if Kp != K:
        a = jnp.pad(a, ((0, 0), (0, Kp - K)))
    nk = Kp // tk
    nn = Np // tn

    def body(a_ref, w_ref, b_ref, o_ref, acc_ref):
        @pl.when(pl.program_id(1) == 0)
        def _init():
            acc_ref[...] = jnp.zeros_like(acc_ref)

        acc_ref[...] += jnp.dot(
            a_ref[...], w_ref[...], preferred_element_type=jnp.float32
        )

        @pl.when(pl.program_id(1) == nk - 1)
        def _fin():
            r = acc_ref[...] + b_ref[...]
            if relu:
                r = jnp.maximum(r, 0.0)
            o_ref[...] = r.astype(o_ref.dtype)

    out = pl.pallas_call(
        body,
        out_shape=jax.ShapeDtypeStruct((M, Np), out_dtype),
        grid=(nn, nk),
        in_specs=[
            pl.BlockSpec((M, tk), lambda j, kk: (0, kk)),
            pl.BlockSpec((tk, tn), lambda j, kk: (kk, j)),
            pl.BlockSpec((1, tn), lambda j, kk: (0, j)),
        ],
        out_specs=pl.BlockSpec((M, tn), lambda j, kk: (0, j)),
        scratch_shapes=[pltpu.VMEM((M, tn), jnp.float32)],
        compiler_params=pltpu.CompilerParams(
            dimension_semantics=("parallel", "arbitrary"),
            vmem_limit_bytes=48 * 1024 * 1024,
        ),
    )(a, w, bias)
    return out[:, :n_out]


def kernel(x, conv1_w, conv1_b, conv2_w, conv2_b, conv3_w, conv3_b,
           conv4_w, conv4_b, conv5_w, conv5_b,
           fc1_w, fc1_b, fc2_w, fc2_b, fc3_w, fc3_b):
    B = x.shape[0]

    xb = jnp.transpose(x.astype(jnp.bfloat16), (0, 2, 3, 1))
    xp = jnp.pad(xb, ((0, 0), (2, 2), (2, 2), (0, 0)))
    xs = jax.lax.conv_general_dilated_patches(
        xp, filter_shape=(4, 4), window_strides=(4, 4), padding="VALID",
        dimension_numbers=("NHWC", "HWIO", "NHWC"))
    x1 = xs.reshape(B * 57 * 57, 48)

    w1 = conv1_w[:363, :64].reshape(3, 11, 11, 64)
    w1 = jnp.pad(w1, ((0, 0), (0, 1), (0, 1), (0, 0)))
    w1 = w1.reshape(3, 3, 4, 3, 4, 64).transpose(1, 3, 0, 2, 4, 5)
    w1 = w1.reshape(9, 48, 64)

    o1 = _conv_flat(x1, w1, conv1_b[:, :64], Wp=57, k=3, pad=0,
                    S=57 * 57, tm_target=512, mask=False)
    p1 = _pool_flat(o1[:B * 3249], Wp=57, tm_target=1024)
    p1 = p1[:B * 3249].reshape(B, 57, 57, 64)[:, 0:53:2, 0:53:2, :]

    x2 = jnp.pad(p1, ((0, 0), (2, 2), (2, 2), (0, 0))).reshape(B * 961, 64)
    w2 = conv2_w[:1600, :192].reshape(64, 5, 5, 192).transpose(1, 2, 0, 3)
    w2 = w2.reshape(25, 64, 192)
    o2 = _conv_flat(x2, w2, conv2_b[:, :192], Wp=31, k=5, pad=2,
                    S=961, tm_target=512, mask=True)
    p2 = _pool_flat(o2[:B * 961], Wp=31, tm_target=1024)
    p2 = p2[:B * 961].reshape(B, 31, 31, 192)[:, 2:27:2, 2:27:2, :]

    x3 = jnp.pad(p2, ((0, 0), (1, 1), (1, 1), (0, 0))).reshape(B * 225, 192)
    w3 = conv3_w[:1728, :384].reshape(192, 3, 3, 384).transpose(1, 2, 0, 3)
    w3 = w3.reshape(9, 192, 384)
    o3 = _conv_flat(x3, w3, conv3_b[:, :384], Wp=15, k=3, pad=1,
                    S=225, tm_target=512, mask=True)

    w4 = conv4_w[:3456, :256].reshape(384, 3, 3, 256).transpose(1, 2, 0, 3)
    w4 = w4.reshape(9, 384, 256)
    o4 = _conv_flat(o3[:B * 225], w4, conv4_b[:, :256], Wp=15, k=3, pad=1,
                    S=225, tm_target=512, mask=True)

    w5 = conv5_w[:2304, :256].reshape(256, 3, 3, 256).transpose(1, 2, 0, 3)
    w5 = w5.reshape(9, 256, 256)
    o5 = _conv_flat(o4[:B * 225], w5, conv5_b[:, :256], Wp=15, k=3, pad=1,
                    S=225, tm_target=512, mask=False)
    p5 = _pool_flat(o5[:B * 225], Wp=15, tm_target=1024)
    p5 = p5[:B * 225].reshape(B, 15, 15, 256)[:, 1:12:2, 1:12:2, :]

    a = p5.reshape(B, 6 * 6 * 256)
    h = _fc(a, fc1_w, fc1_b, tn=2048, tk=2304, relu=True,
            n_out=4096, out_dtype=jnp.bfloat16)
    h = _fc(h, fc2_w, fc2_b, tn=2048, tk=2048, relu=True,
            n_out=4096, out_dtype=jnp.bfloat16)
    out = _fc(h, fc3_w, fc3_b, tn=128, tk=4096, relu=False,
              n_out=_NUM_CLASSES, out_dtype=jnp.float32)
    return out


# speedup vs baseline: 1.0001x; 1.0001x over previous
import jax
import jax.numpy as jnp
from jax.experimental import pallas as pl
from jax.experimental.pallas import tpu as pltpu

_NUM_CLASSES = 100


def _ru(x, m):
    return ((x + m - 1) // m) * m


def _conv_flat(x_flat, w_taps, bias, *, Wp, k, pad, S, tm_target, mask):
    R, Cin = x_flat.shape
    T, _, Cout = w_taps.shape
    offsets = [dh * Wp + dw for dh in range(k) for dw in range(k)]
    D = pad * Wp + pad
    tm = min(tm_target, _ru(R, 8))
    nb = -(-R // tm)
    Rp = nb * tm
    ext_len = (nb + 1) * tm
    x_ext = jnp.pad(x_flat, ((D, ext_len - D - R), (0, 0)))
    halo = _ru(max(offsets), 8) if max(offsets) else 0
    Hp = S // Wp

    def body(x0_ref, x1_ref, w_ref, b_ref, o_ref):
        if halo:
            xcat = jnp.concatenate([x0_ref[...], x1_ref[:halo]], axis=0)
        else:
            xcat = x0_ref[...]
        acc = jnp.zeros((tm, Cout), jnp.float32)
        for t, off in enumerate(offsets):
            acc = acc + jnp.dot(
                xcat[off:off + tm], w_ref[t], preferred_element_type=jnp.float32
            )
        r = jnp.maximum(acc + b_ref[...], 0.0)
        if mask:
            row = pl.program_id(0) * tm + jax.lax.broadcasted_iota(
                jnp.int32, (tm, 1), 0
            )
            s = jax.lax.rem(row, S)
            h = jax.lax.div(s, Wp)
            w = jax.lax.rem(s, Wp)
            ok = (h >= pad) & (h < Hp - pad) & (w >= pad) & (w < Wp - pad)
            r = jnp.where(ok, r, 0.0)
        o_ref[...] = r.astype(o_ref.dtype)

    return pl.pallas_call(
        body,
        out_shape=jax.ShapeDtypeStruct((Rp, Cout), jnp.bfloat16),
        grid=(nb,),
        in_specs=[
            pl.BlockSpec((tm, Cin), lambda i: (i, 0)),
            pl.BlockSpec((tm, Cin), lambda i: (i + 1, 0)),
            pl.BlockSpec((T, Cin, Cout), lambda i: (0, 0, 0)),
            pl.BlockSpec((1, Cout), lambda i: (0, 0)),
        ],
        out_specs=pl.BlockSpec((tm, Cout), lambda i: (i, 0)),
        compiler_params=pltpu.CompilerParams(
            dimension_semantics=("parallel",),
            vmem_limit_bytes=48 * 1024 * 1024,
        ),
    )(x_ext, x_ext, w_taps, bias)


def _pool_flat(x_flat, *, Wp, tm_target):
    R, C = x_flat.shape
    offsets = [dh * Wp + dw for dh in range(3) for dw in range(3)]
    tm = min(tm_target, _ru(R, 8))
    nb = -(-R // tm)
    Rp = nb * tm
    ext_len = (nb + 1) * tm
    x_ext = jnp.pad(x_flat, ((0, ext_len - R), (0, 0)))
    halo = _ru(max(offsets), 8)

    def body(x0_ref, x1_ref, o_ref):
        xcat = jnp.concatenate([x0_ref[...], x1_ref[:halo]], axis=0)
        r = xcat[0:tm]
        for off in offsets[1:]:
            r = jnp.maximum(r, xcat[off:off + tm])
        o_ref[...] = r

    return pl.pallas_call(
        body,
        out_shape=jax.ShapeDtypeStruct((Rp, C), x_flat.dtype),
        grid=(nb,),
        in_specs=[
            pl.BlockSpec((tm, C), lambda i: (i, 0)),
            pl.BlockSpec((tm, C), lambda i: (i + 1, 0)),
        ],
        out_specs=pl.BlockSpec((tm, C), lambda i: (i, 0)),
        compiler_params=pltpu.CompilerParams(
            dimension_semantics=("parallel",),
            vmem_limit_bytes=48 * 1024 * 1024,
        ),
    )(x_ext, x_ext)


def _fc(a, w, bias, *, tn, tk, relu, n_out, out_dtype):
    M, K = a.shape
    Kp, Np = w.shape
    if Kp != K:
        a = jnp.pad(a, ((0, 0), (0, Kp - K)))
    nk = Kp // tk
    nn = Np // tn

    def body(a_ref, w_ref, b_ref, o_ref, acc_ref):
        @pl.when(pl.program_id(1) == 0)
        def _init():
            acc_ref[...] = jnp.zeros_like(acc_ref)

        acc_ref[...] += jnp.dot(
            a_ref[...], w_ref[...], preferred_element_type=jnp.float32
        )

        @pl.when(pl.program_id(1) == nk - 1)
        def _fin():
            r = acc_ref[...] + b_ref[...]
            if relu:
                r = jnp.maximum(r, 0.0)
            o_ref[...] = r.astype(o_ref.dtype)

    out = pl.pallas_call(
        body,
        out_shape=jax.ShapeDtypeStruct((M, Np), out_dtype),
        grid=(nn, nk),
        in_specs=[
            pl.BlockSpec((M, tk), lambda j, kk: (0, kk)),
            pl.BlockSpec((tk, tn), lambda j, kk: (kk, j)),
            pl.BlockSpec((1, tn), lambda j, kk: (0, j)),
        ],
        out_specs=pl.BlockSpec((M, tn), lambda j, kk: (0, j)),
        scratch_shapes=[pltpu.VMEM((M, tn), jnp.float32)],
        compiler_params=pltpu.CompilerParams(
            dimension_semantics=("parallel", "arbitrary"),
            vmem_limit_bytes=48 * 1024 * 1024,
        ),
    )(a, w, bias)
    return out[:, :n_out]


def kernel(x, conv1_w, conv1_b, conv2_w, conv2_b, conv3_w, conv3_b,
           conv4_w, conv4_b, conv5_w, conv5_b,
           fc1_w, fc1_b, fc2_w, fc2_b, fc3_w, fc3_b):
    B = x.shape[0]

    eye = jnp.eye(48, dtype=jnp.bfloat16).reshape(3, 4, 4, 48)
    eye = eye.transpose(1, 2, 0, 3)
    xs = jax.lax.conv_general_dilated(
        x.astype(jnp.bfloat16), eye, window_strides=(4, 4),
        padding=((2, 2), (2, 2)),
        dimension_numbers=("NCHW", "HWIO", "NHWC"),
        preferred_element_type=jnp.bfloat16)
    x1 = xs.reshape(B * 57 * 57, 48)

    w1 = conv1_w[:363, :64].reshape(3, 11, 11, 64)
    w1 = jnp.pad(w1, ((0, 0), (0, 1), (0, 1), (0, 0)))
    w1 = w1.reshape(3, 3, 4, 3, 4, 64).transpose(1, 3, 0, 2, 4, 5)
    w1 = w1.reshape(9, 48, 64)

    o1 = _conv_flat(x1, w1, conv1_b[:, :64], Wp=57, k=3, pad=0,
                    S=57 * 57, tm_target=512, mask=False)
    p1 = _pool_flat(o1[:B * 3249], Wp=57, tm_target=1024)
    p1 = p1[:B * 3249].reshape(B, 57, 57, 64)[:, 0:53:2, 0:53:2, :]

    x2 = jnp.pad(p1, ((0, 0), (2, 2), (2, 2), (0, 0))).reshape(B * 961, 64)
    w2 = conv2_w[:1600, :192].reshape(64, 5, 5, 192).transpose(1, 2, 0, 3)
    w2 = w2.reshape(25, 64, 192)
    o2 = _conv_flat(x2, w2, conv2_b[:, :192], Wp=31, k=5, pad=2,
                    S=961, tm_target=512, mask=True)
    p2 = _pool_flat(o2[:B * 961], Wp=31, tm_target=1024)
    p2 = p2[:B * 961].reshape(B, 31, 31, 192)[:, 2:27:2, 2:27:2, :]

    x3 = jnp.pad(p2, ((0, 0), (1, 1), (1, 1), (0, 0))).reshape(B * 225, 192)
    w3 = conv3_w[:1728, :384].reshape(192, 3, 3, 384).transpose(1, 2, 0, 3)
    w3 = w3.reshape(9, 192, 384)
    o3 = _conv_flat(x3, w3, conv3_b[:, :384], Wp=15, k=3, pad=1,
                    S=225, tm_target=512, mask=True)

    w4 = conv4_w[:3456, :256].reshape(384, 3, 3, 256).transpose(1, 2, 0, 3)
    w4 = w4.reshape(9, 384, 256)
    o4 = _conv_flat(o3[:B * 225], w4, conv4_b[:, :256], Wp=15, k=3, pad=1,
                    S=225, tm_target=512, mask=True)

    w5 = conv5_w[:2304, :256].reshape(256, 3, 3, 256).transpose(1, 2, 0, 3)
    w5 = w5.reshape(9, 256, 256)
    o5 = _conv_flat(o4[:B * 225], w5, conv5_b[:, :256], Wp=15, k=3, pad=1,
                    S=225, tm_target=512, mask=False)
    p5 = _pool_flat(o5[:B * 225], Wp=15, tm_target=1024)
    p5 = p5[:B * 225].reshape(B, 15, 15, 256)[:, 1:12:2, 1:12:2, :]

    a = p5.reshape(B, 6 * 6 * 256)
    h = _fc(a, fc1_w, fc1_b, tn=2048, tk=2304, relu=True,
            n_out=4096, out_dtype=jnp.bfloat16)
    h = _fc(h, fc2_w, fc2_b, tn=2048, tk=2048, relu=True,
            n_out=4096, out_dtype=jnp.bfloat16)
    out = _fc(h, fc3_w, fc3_b, tn=128, tk=4096, relu=False,
              n_out=_NUM_CLASSES, out_dtype=jnp.float32)
    return out


# speedup vs baseline: 1.4750x; 1.4748x over previous
import jax
import jax.numpy as jnp
from jax.experimental import pallas as pl
from jax.experimental.pallas import tpu as pltpu

_NUM_CLASSES = 100


def _ru(x, m):
    return ((x + m - 1) // m) * m


def _conv_flat(x_flat, w_taps, bias, *, Wp, k, pad, S, tm_target, mask):
    R, Cin = x_flat.shape
    T, _, Cout = w_taps.shape
    offsets = [dh * Wp + dw for dh in range(k) for dw in range(k)]
    D = pad * Wp + pad
    tm = min(tm_target, _ru(R, 8))
    nb = -(-R // tm)
    Rp = nb * tm
    ext_len = (nb + 1) * tm
    x_ext = jnp.pad(x_flat, ((D, ext_len - D - R), (0, 0)))
    halo = _ru(max(offsets), 8) if max(offsets) else 0
    Hp = S // Wp

    def body(x0_ref, x1_ref, w_ref, b_ref, o_ref):
        if halo:
            xcat = jnp.concatenate([x0_ref[...], x1_ref[:halo]], axis=0)
        else:
            xcat = x0_ref[...]
        acc = jnp.zeros((tm, Cout), jnp.float32)
        for t, off in enumerate(offsets):
            acc = acc + jnp.dot(
                xcat[off:off + tm], w_ref[t], preferred_element_type=jnp.float32
            )
        r = jnp.maximum(acc + b_ref[...], 0.0)
        if mask:
            row = pl.program_id(0) * tm + jax.lax.broadcasted_iota(
                jnp.int32, (tm, 1), 0
            )
            s = jax.lax.rem(row, S)
            h = jax.lax.div(s, Wp)
            w = jax.lax.rem(s, Wp)
            ok = (h >= pad) & (h < Hp - pad) & (w >= pad) & (w < Wp - pad)
            r = jnp.where(ok, r, 0.0)
        o_ref[...] = r.astype(o_ref.dtype)

    return pl.pallas_call(
        body,
        out_shape=jax.ShapeDtypeStruct((Rp, Cout), jnp.bfloat16),
        grid=(nb,),
        in_specs=[
            pl.BlockSpec((tm, Cin), lambda i: (i, 0)),
            pl.BlockSpec((tm, Cin), lambda i: (i + 1, 0)),
            pl.BlockSpec((T, Cin, Cout), lambda i: (0, 0, 0)),
            pl.BlockSpec((1, Cout), lambda i: (0, 0)),
        ],
        out_specs=pl.BlockSpec((tm, Cout), lambda i: (i, 0)),
        compiler_params=pltpu.CompilerParams(
            dimension_semantics=("parallel",),
            vmem_limit_bytes=48 * 1024 * 1024,
        ),
    )(x_ext, x_ext, w_taps, bias)


def _pool_flat(x_flat, *, Wp, tm_target):
    R, C = x_flat.shape
    offsets = [dh * Wp + dw for dh in range(3) for dw in range(3)]
    tm = min(tm_target, _ru(R, 8))
    nb = -(-R // tm)
    Rp = nb * tm
    ext_len = (nb + 1) * tm
    x_ext = jnp.pad(x_flat, ((0, ext_len - R), (0, 0)))
    halo = _ru(max(offsets), 8)

    def body(x0_ref, x1_ref, o_ref):
        xcat = jnp.concatenate([x0_ref[...], x1_ref[:halo]], axis=0)
        r = xcat[0:tm]
        for off in offsets[1:]:
            r = jnp.maximum(r, xcat[off:off + tm])
        o_ref[...] = r

    return pl.pallas_call(
        body,
        out_shape=jax.ShapeDtypeStruct((Rp, C), x_flat.dtype),
        grid=(nb,),
        in_specs=[
            pl.BlockSpec((tm, C), lambda i: (i, 0)),
            pl.BlockSpec((tm, C), lambda i: (i + 1, 0)),
        ],
        out_specs=pl.BlockSpec((tm, C), lambda i: (i, 0)),
        compiler_params=pltpu.CompilerParams(
            dimension_semantics=("parallel",),
            vmem_limit_bytes=48 * 1024 * 1024,
        ),
    )(x_ext, x_ext)


def _fc(a, w, bias, *, tn, tk, relu, n_out, out_dtype):
    M, K = a.shape
    Kp, Np = w.shape
    if Kp != K:
        a = jnp.pad(a, ((0, 0), (0, Kp - K)))
    nk = Kp // tk
    nn = Np // tn

    def body(a_ref, w_ref, b_ref, o_ref, acc_ref):
        @pl.when(pl.program_id(1) == 0)
        def _init():
            acc_ref[...] = jnp.zeros_like(acc_ref)

        acc_ref[...] += jnp.dot(
            a_ref[...], w_ref[...], preferred_element_type=jnp.float32
        )

        @pl.when(pl.program_id(1) == nk - 1)
        def _fin():
            r = acc_ref[...] + b_ref[...]
            if relu:
                r = jnp.maximum(r, 0.0)
            o_ref[...] = r.astype(o_ref.dtype)

    out = pl.pallas_call(
        body,
        out_shape=jax.ShapeDtypeStruct((M, Np), out_dtype),
        grid=(nn, nk),
        in_specs=[
            pl.BlockSpec((M, tk), lambda j, kk: (0, kk)),
            pl.BlockSpec((tk, tn), lambda j, kk: (kk, j)),
            pl.BlockSpec((1, tn), lambda j, kk: (0, j)),
        ],
        out_specs=pl.BlockSpec((M, tn), lambda j, kk: (0, j)),
        scratch_shapes=[pltpu.VMEM((M, tn), jnp.float32)],
        compiler_params=pltpu.CompilerParams(
            dimension_semantics=("parallel", "arbitrary"),
            vmem_limit_bytes=48 * 1024 * 1024,
        ),
    )(a, w, bias)
    return out[:, :n_out]


def _s2d(xp):
    B = xp.shape[0]
    xs = xp.reshape(B, 3, 57, 4, 57, 4).transpose(0, 2, 4, 3, 5, 1)
    return xs.reshape(B * 57 * 57, 48)


def kernel(x, conv1_w, conv1_b, conv2_w, conv2_b, conv3_w, conv3_b,
           conv4_w, conv4_b, conv5_w, conv5_b,
           fc1_w, fc1_b, fc2_w, fc2_b, fc3_w, fc3_b):
    B = x.shape[0]

    xp = jnp.pad(x.astype(jnp.bfloat16), ((0, 0), (0, 0), (2, 2), (2, 2)))
    x1 = _s2d(xp)

    w1 = conv1_w[:363, :64].reshape(3, 11, 11, 64)
    w1 = jnp.pad(w1, ((0, 0), (0, 1), (0, 1), (0, 0)))
    w1 = w1.reshape(3, 3, 4, 3, 4, 64).transpose(1, 3, 2, 4, 0, 5)
    w1 = w1.reshape(9, 48, 64)

    o1 = _conv_flat(x1, w1, conv1_b[:, :64], Wp=57, k=3, pad=0,
                    S=57 * 57, tm_target=512, mask=False)
    p1 = _pool_flat(o1[:B * 3249], Wp=57, tm_target=1024)
    p1 = p1[:B * 3249].reshape(B, 57, 57, 64)[:, 0:53:2, 0:53:2, :]

    x2 = jnp.pad(p1, ((0, 0), (2, 2), (2, 2), (0, 0))).reshape(B * 961, 64)
    w2 = conv2_w[:1600, :192].reshape(64, 5, 5, 192).transpose(1, 2, 0, 3)
    w2 = w2.reshape(25, 64, 192)
    o2 = _conv_flat(x2, w2, conv2_b[:, :192], Wp=31, k=5, pad=2,
                    S=961, tm_target=512, mask=True)
    p2 = _pool_flat(o2[:B * 961], Wp=31, tm_target=1024)
    p2 = p2[:B * 961].reshape(B, 31, 31, 192)[:, 2:27:2, 2:27:2, :]

    x3 = jnp.pad(p2, ((0, 0), (1, 1), (1, 1), (0, 0))).reshape(B * 225, 192)
    w3 = conv3_w[:1728, :384].reshape(192, 3, 3, 384).transpose(1, 2, 0, 3)
    w3 = w3.reshape(9, 192, 384)
    o3 = _conv_flat(x3, w3, conv3_b[:, :384], Wp=15, k=3, pad=1,
                    S=225, tm_target=512, mask=True)

    w4 = conv4_w[:3456, :256].reshape(384, 3, 3, 256).transpose(1, 2, 0, 3)
    w4 = w4.reshape(9, 384, 256)
    o4 = _conv_flat(o3[:B * 225], w4, conv4_b[:, :256], Wp=15, k=3, pad=1,
                    S=225, tm_target=512, mask=True)

    w5 = conv5_w[:2304, :256].reshape(256, 3, 3, 256).transpose(1, 2, 0, 3)
    w5 = w5.reshape(9, 256, 256)
    o5 = _conv_flat(o4[:B * 225], w5, conv5_b[:, :256], Wp=15, k=3, pad=1,
                    S=225, tm_target=512, mask=False)
    p5 = _pool_flat(o5[:B * 225], Wp=15, tm_target=1024)
    p5 = p5[:B * 225].reshape(B, 15, 15, 256)[:, 1:12:2, 1:12:2, :]

    a = p5.reshape(B, 6 * 6 * 256)
    h = _fc(a, fc1_w, fc1_b, tn=2048, tk=2304, relu=True,
            n_out=4096, out_dtype=jnp.bfloat16)
    h = _fc(h, fc2_w, fc2_b, tn=2048, tk=2048, relu=True,
            n_out=4096, out_dtype=jnp.bfloat16)
    out = _fc(h, fc3_w, fc3_b, tn=128, tk=4096, relu=False,
              n_out=_NUM_CLASSES, out_dtype=jnp.float32)
    return out


# speedup vs baseline: 3.3635x; 2.2804x over previous
import jax
import jax.numpy as jnp
from jax.experimental import pallas as pl
from jax.experimental.pallas import tpu as pltpu

_NUM_CLASSES = 100


def _ru(x, m):
    return ((x + m - 1) // m) * m


def _conv_flat(x_flat, w_taps, bias, *, Wp, k, pad, S, tm_target, mask):
    R, Cin = x_flat.shape
    T, _, Cout = w_taps.shape
    offsets = [dh * Wp + dw for dh in range(k) for dw in range(k)]
    D = pad * Wp + pad
    tm = min(tm_target, _ru(R, 8))
    nb = -(-R // tm)
    Rp = nb * tm
    ext_len = (nb + 1) * tm
    x_ext = jnp.pad(x_flat, ((D, ext_len - D - R), (0, 0)))
    halo = _ru(max(offsets), 8) if max(offsets) else 0
    Hp = S // Wp

    def body(x0_ref, x1_ref, w_ref, b_ref, o_ref):
        if halo:
            xcat = jnp.concatenate([x0_ref[...], x1_ref[:halo]], axis=0)
        else:
            xcat = x0_ref[...]
        acc = jnp.zeros((tm, Cout), jnp.float32)
        for t, off in enumerate(offsets):
            acc = acc + jnp.dot(
                xcat[off:off + tm], w_ref[t], preferred_element_type=jnp.float32
            )
        r = jnp.maximum(acc + b_ref[...], 0.0)
        if mask:
            row = pl.program_id(0) * tm + jax.lax.broadcasted_iota(
                jnp.int32, (tm, 1), 0
            )
            s = jax.lax.rem(row, S)
            h = jax.lax.div(s, Wp)
            w = jax.lax.rem(s, Wp)
            ok = (h >= pad) & (h < Hp - pad) & (w >= pad) & (w < Wp - pad)
            r = jnp.where(ok, r, 0.0)
        o_ref[...] = r.astype(o_ref.dtype)

    return pl.pallas_call(
        body,
        out_shape=jax.ShapeDtypeStruct((Rp, Cout), jnp.bfloat16),
        grid=(nb,),
        in_specs=[
            pl.BlockSpec((tm, Cin), lambda i: (i, 0)),
            pl.BlockSpec((tm, Cin), lambda i: (i + 1, 0)),
            pl.BlockSpec((T, Cin, Cout), lambda i: (0, 0, 0)),
            pl.BlockSpec((1, Cout), lambda i: (0, 0)),
        ],
        out_specs=pl.BlockSpec((tm, Cout), lambda i: (i, 0)),
        compiler_params=pltpu.CompilerParams(
            dimension_semantics=("parallel",),
            vmem_limit_bytes=48 * 1024 * 1024,
        ),
    )(x_ext, x_ext, w_taps, bias)


def _pool_flat(x_flat, *, Wp, tm_target):
    R, C = x_flat.shape
    offsets = [dh * Wp + dw for dh in range(3) for dw in range(3)]
    tm = min(tm_target, _ru(R, 8))
    nb = -(-R // tm)
    Rp = nb * tm
    ext_len = (nb + 1) * tm
    x_ext = jnp.pad(x_flat, ((0, ext_len - R), (0, 0)))
    halo = _ru(max(offsets), 8)

    def body(x0_ref, x1_ref, o_ref):
        xcat = jnp.concatenate([x0_ref[...], x1_ref[:halo]], axis=0)
        r = xcat[0:tm]
        for off in offsets[1:]:
            r = jnp.maximum(r, xcat[off:off + tm])
        o_ref[...] = r

    return pl.pallas_call(
        body,
        out_shape=jax.ShapeDtypeStruct((Rp, C), x_flat.dtype),
        grid=(nb,),
        in_specs=[
            pl.BlockSpec((tm, C), lambda i: (i, 0)),
            pl.BlockSpec((tm, C), lambda i: (i + 1, 0)),
        ],
        out_specs=pl.BlockSpec((tm, C), lambda i: (i, 0)),
        compiler_params=pltpu.CompilerParams(
            dimension_semantics=("parallel",),
            vmem_limit_bytes=48 * 1024 * 1024,
        ),
    )(x_ext, x_ext)


def _fc(a, w, bias, *, tn, tk, relu, n_out, out_dtype):
    M, K = a.shape
    Kp, Np = w.shape
    if Kp != K:
        a = jnp.pad(a, ((0, 0), (0, Kp - K)))
    nk = Kp // tk
    nn = Np // tn

    def body(a_ref, w_ref, b_ref, o_ref, acc_ref):
        @pl.when(pl.program_id(1) == 0)
        def _init():
            acc_ref[...] = jnp.zeros_like(acc_ref)

        acc_ref[...] += jnp.dot(
            a_ref[...], w_ref[...], preferred_element_type=jnp.float32
        )

        @pl.when(pl.program_id(1) == nk - 1)
        def _fin():
            r = acc_ref[...] + b_ref[...]
            if relu:
                r = jnp.maximum(r, 0.0)
            o_ref[...] = r.astype(o_ref.dtype)

    out = pl.pallas_call(
        body,
        out_shape=jax.ShapeDtypeStruct((M, Np), out_dtype),
        grid=(nn, nk),
        in_specs=[
            pl.BlockSpec((M, tk), lambda j, kk: (0, kk)),
            pl.BlockSpec((tk, tn), lambda j, kk: (kk, j)),
            pl.BlockSpec((1, tn), lambda j, kk: (0, j)),
        ],
        out_specs=pl.BlockSpec((M, tn), lambda j, kk: (0, j)),
        scratch_shapes=[pltpu.VMEM((M, tn), jnp.float32)],
        compiler_params=pltpu.CompilerParams(
            dimension_semantics=("parallel", "arbitrary"),
            vmem_limit_bytes=48 * 1024 * 1024,
        ),
    )(a, w, bias)
    return out[:, :n_out]


def _s2d(xp):
    B = xp.shape[0]
    xs = xp.reshape(B, 3, 57, 4, 57, 4).transpose(0, 2, 4, 3, 5, 1)
    return xs.reshape(B * 57 * 57, 48)


def _features(xp, w1, b1, w2, b2, w3, b3, w4, b4, w5, b5):
    B = xp.shape[0]
    x1 = _s2d(xp)
    o1 = _conv_flat(x1, w1, b1, Wp=57, k=3, pad=0,
                    S=57 * 57, tm_target=512, mask=False)
    p1 = _pool_flat(o1[:B * 3249], Wp=57, tm_target=1024)
    p1 = p1[:B * 3249].reshape(B, 57, 57, 64)[:, 0:53:2, 0:53:2, :]

    x2 = jnp.pad(p1, ((0, 0), (2, 2), (2, 2), (0, 0))).reshape(B * 961, 64)
    o2 = _conv_flat(x2, w2, b2, Wp=31, k=5, pad=2,
                    S=961, tm_target=512, mask=True)
    p2 = _pool_flat(o2[:B * 961], Wp=31, tm_target=1024)
    p2 = p2[:B * 961].reshape(B, 31, 31, 192)[:, 2:27:2, 2:27:2, :]

    x3 = jnp.pad(p2, ((0, 0), (1, 1), (1, 1), (0, 0))).reshape(B * 225, 192)
    o3 = _conv_flat(x3, w3, b3, Wp=15, k=3, pad=1,
                    S=225, tm_target=512, mask=True)
    o4 = _conv_flat(o3[:B * 225], w4, b4, Wp=15, k=3, pad=1,
                    S=225, tm_target=512, mask=True)
    o5 = _conv_flat(o4[:B * 225], w5, b5, Wp=15, k=3, pad=1,
                    S=225, tm_target=512, mask=False)
    p5 = _pool_flat(o5[:B * 225], Wp=15, tm_target=1024)
    p5 = p5[:B * 225].reshape(B, 15, 15, 256)[:, 1:12:2, 1:12:2, :]
    return p5.reshape(B, 6 * 6 * 256)


def kernel(x, conv1_w, conv1_b, conv2_w, conv2_b, conv3_w, conv3_b,
           conv4_w, conv4_b, conv5_w, conv5_b,
           fc1_w, fc1_b, fc2_w, fc2_b, fc3_w, fc3_b):
    B = x.shape[0]

    xp = jnp.pad(x.astype(jnp.bfloat16), ((0, 0), (0, 0), (2, 2), (2, 2)))

    w1 = conv1_w[:363, :64].reshape(3, 11, 11, 64)
    w1 = jnp.pad(w1, ((0, 0), (0, 1), (0, 1), (0, 0)))
    w1 = w1.reshape(3, 3, 4, 3, 4, 64).transpose(1, 3, 2, 4, 0, 5)
    w1 = w1.reshape(9, 48, 64)

    w2 = conv2_w[:1600, :192].reshape(64, 5, 5, 192).transpose(1, 2, 0, 3)
    w2 = w2.reshape(25, 64, 192)
    w3 = conv3_w[:1728, :384].reshape(192, 3, 3, 384).transpose(1, 2, 0, 3)
    w3 = w3.reshape(9, 192, 384)
    w4 = conv4_w[:3456, :256].reshape(384, 3, 3, 256).transpose(1, 2, 0, 3)
    w4 = w4.reshape(9, 384, 256)
    w5 = conv5_w[:2304, :256].reshape(256, 3, 3, 256).transpose(1, 2, 0, 3)
    w5 = w5.reshape(9, 256, 256)

    nchunk = 4
    Bc = B // nchunk if B % 4 == 0 else B
    feats = [
        _features(xp[i:i + Bc], w1, conv1_b[:, :64], w2, conv2_b[:, :192],
                  w3, conv3_b[:, :384], w4, conv4_b[:, :256],
                  w5, conv5_b[:, :256])
        for i in range(0, B, Bc)
    ]
    a = jnp.concatenate(feats, axis=0)

    h = _fc(a, fc1_w, fc1_b, tn=2048, tk=2304, relu=True,
            n_out=4096, out_dtype=jnp.bfloat16)
    h = _fc(h, fc2_w, fc2_b, tn=2048, tk=2048, relu=True,
            n_out=4096, out_dtype=jnp.bfloat16)
    out = _fc(h, fc3_w, fc3_b, tn=128, tk=4096, relu=False,
              n_out=_NUM_CLASSES, out_dtype=jnp.float32)
    return out
